# baseline (speedup 1.0000x reference)
import functools

import jax
import jax.numpy as jnp
from jax import lax
from jax.experimental import pallas as pl
from jax.experimental.pallas import tpu as pltpu
from jax.experimental.pallas import tpu_sc as plsc

B = 16384
V = 100000
D = 128

_NC = 2
_NS = 16
_NW = _NC * _NS

_NCHUNKS = 4
_CB = B // _NCHUNKS


def _sc_gather3(t0, t1, t2, i0, i1, i2, nrows, row0):
    bpw = nrows // _NW
    sub = max(1, bpw // 256)
    ch = bpw // sub
    nbuf = 3
    lookahead = 2

    mesh = plsc.VectorSubcoreMesh(
        core_axis_name="c", subcore_axis_name="s",
        num_cores=_NC, num_subcores=_NS)

    @functools.partial(
        pl.kernel,
        out_type=(
            jax.ShapeDtypeStruct((nrows, D), jnp.float32),
            jax.ShapeDtypeStruct((nrows, D), jnp.float32),
            jax.ShapeDtypeStruct((nrows, D), jnp.float32),
        ),
        mesh=mesh,
        scratch_types=(
            [pltpu.VMEM((ch,), jnp.int32) for _ in range(nbuf)]
            + [pltpu.VMEM((ch, D), jnp.float32) for _ in range(nbuf)]
            + [pltpu.SemaphoreType.DMA for _ in range(2 * nbuf)]
        ),
    )
    def gather_kernel(t0_h, t1_h, t2_h, i0_h, i1_h, i2_h,
                      o0_h, o1_h, o2_h, *scr):
        idxs = scr[:nbuf]
        bufs = scr[nbuf:2 * nbuf]
        sems_g = scr[2 * nbuf:3 * nbuf]
        sems_w = scr[3 * nbuf:]
        wid = lax.axis_index("s") * _NC + lax.axis_index("c")
        base = wid * bpw
        ibase = row0 + wid * bpw
        tasks = [(tab, idx, out, h * ch)
                 for (tab, idx, out) in ((t0_h, i0_h, o0_h),
                                         (t1_h, i1_h, o1_h),
                                         (t2_h, i2_h, o2_h))
                 for h in range(sub)]
        n = len(tasks)
        gath = [None] * nbuf
        writes = [None] * nbuf
        for t in range(n + lookahead):
            if t < n:
                slot = t % nbuf
                if writes[slot] is not None:
                    writes[slot].wait()
                    writes[slot] = None
                tab, idx, out, off = tasks[t]
                pltpu.sync_copy(idx.at[pl.ds(ibase + off, ch)], idxs[slot])
                gath[slot] = pltpu.async_copy(
                    tab.at[idxs[slot]], bufs[slot], sems_g[slot])
            if t >= lookahead:
                u = t - lookahead
                slot = u % nbuf
                gath[slot].wait()
                _, _, out, off = tasks[u]
                writes[slot] = pltpu.async_copy(
                    bufs[slot], out.at[pl.ds(base + off, ch)], sems_w[slot])
        for w in writes:
            if w is not None:
                w.wait()

    return gather_kernel(t0, t1, t2, i0, i1, i2)


_TCG = 2


def _tc_body(e0_r, e1_r, e2_r, w0_r, w1_r, w2_r, b_r, out_r):
    acc = jnp.dot(e0_r[...], w0_r[...], preferred_element_type=jnp.float32)
    acc += jnp.dot(e1_r[...], w1_r[...], preferred_element_type=jnp.float32)
    acc += jnp.dot(e2_r[...], w2_r[...], preferred_element_type=jnp.float32)
    out_r[...] = acc + b_r[...]


def _tc_body_alias(e0_r, e1_r, e2_r, w0_r, w1_r, w2_r, b_r, acc_r, out_r):
    del acc_r
    _tc_body(e0_r, e1_r, e2_r, w0_r, w1_r, w2_r, b_r, out_r)


def _tc_matmul_chunk(e0, e1, e2, w0, w1, w2, bsum, chunk, acc):
    bs = _CB // _TCG
    eb = pl.BlockSpec((bs, D), lambda i: (i, 0))
    wb = pl.BlockSpec((D, D), lambda i: (0, 0))
    bb = pl.BlockSpec((1, D), lambda i: (0, 0))
    out_spec = pl.BlockSpec((bs, D), lambda i: (chunk * _TCG + i, 0))
    params = pltpu.CompilerParams(dimension_semantics=("arbitrary",))
    if acc is None:
        return pl.pallas_call(
            _tc_body,
            grid=(_TCG,),
            in_specs=[eb, eb, eb, wb, wb, wb, bb],
            out_specs=out_spec,
            out_shape=jax.ShapeDtypeStruct((B, D), jnp.float32),
            compiler_params=params,
        )(e0, e1, e2, w0, w1, w2, bsum)
    return pl.pallas_call(
        _tc_body_alias,
        grid=(_TCG,),
        in_specs=[eb, eb, eb, wb, wb, wb, bb,
                  pl.BlockSpec(memory_space=pl.ANY)],
        out_specs=out_spec,
        out_shape=jax.ShapeDtypeStruct((B, D), jnp.float32),
        input_output_aliases={7: 0},
        compiler_params=params,
    )(e0, e1, e2, w0, w1, w2, bsum, acc)


def kernel(Target, Type, Location, T_Target, T_Type, T_Location,
           W0, b0, W1, b1, W2, b2):
    i0 = Target.astype(jnp.int32)
    i1 = Type.astype(jnp.int32)
    i2 = Location.astype(jnp.int32)
    bsum = (b0 + b1 + b2).reshape(1, D)
    chunks = []
    for c in range(_NCHUNKS):
        chunks.append(_sc_gather3(T_Target, T_Type, T_Location,
                                  i0, i1, i2, _CB, c * _CB))
    acc = None
    for c, (e0, e1, e2) in enumerate(chunks):
        acc = _tc_matmul_chunk(e0, e1, e2, W0, W1, W2, bsum, c, acc)
    return acc


# speedup vs baseline: 1.1986x; 1.1986x over previous
import functools

import jax
import jax.numpy as jnp
from jax import lax
from jax.experimental import pallas as pl
from jax.experimental.pallas import tpu as pltpu
from jax.experimental.pallas import tpu_sc as plsc

B = 16384
V = 100000
D = 128

_NC = 2
_NS = 16
_NW = _NC * _NS

_NCHUNKS = 2
_CB = B // _NCHUNKS


def _sc_gather3(t0, t1, t2, i0, i1, i2, nrows, row0):
    bpw = nrows // _NW
    sub = max(1, bpw // 256)
    ch = bpw // sub
    nbuf = 3
    lookahead = 2

    mesh = plsc.VectorSubcoreMesh(
        core_axis_name="c", subcore_axis_name="s",
        num_cores=_NC, num_subcores=_NS)

    ntasks = 3 * sub

    @functools.partial(
        pl.kernel,
        out_type=(
            jax.ShapeDtypeStruct((nrows, D), jnp.float32),
            jax.ShapeDtypeStruct((nrows, D), jnp.float32),
            jax.ShapeDtypeStruct((nrows, D), jnp.float32),
        ),
        mesh=mesh,
        scratch_types=(
            [pltpu.VMEM((ch,), jnp.int32) for _ in range(ntasks)]
            + [pltpu.VMEM((ch, D), jnp.float32) for _ in range(nbuf)]
            + [pltpu.SemaphoreType.DMA for _ in range(2 * nbuf + 1)]
        ),
    )
    def gather_kernel(t0_h, t1_h, t2_h, i0_h, i1_h, i2_h,
                      o0_h, o1_h, o2_h, *scr):
        idxs = scr[:ntasks]
        bufs = scr[ntasks:ntasks + nbuf]
        sems_g = scr[ntasks + nbuf:ntasks + 2 * nbuf]
        sems_w = scr[ntasks + 2 * nbuf:ntasks + 3 * nbuf]
        sem_i = scr[ntasks + 3 * nbuf]
        wid = lax.axis_index("s") * _NC + lax.axis_index("c")
        base = wid * bpw
        ibase = row0 + wid * bpw
        tasks = [(tab, idx, out, h * ch)
                 for (tab, idx, out) in ((t0_h, i0_h, o0_h),
                                         (t1_h, i1_h, o1_h),
                                         (t2_h, i2_h, o2_h))
                 for h in range(sub)]
        n = len(tasks)
        icopies = [pltpu.async_copy(idx.at[pl.ds(ibase + off, ch)],
                                    idxs[t], sem_i)
                   for t, (_, idx, _, off) in enumerate(tasks)]
        for d in icopies:
            d.wait()
        gath = [None] * nbuf
        writes = [None] * nbuf
        for t in range(n + lookahead):
            if t < n:
                slot = t % nbuf
                if writes[slot] is not None:
                    writes[slot].wait()
                    writes[slot] = None
                tab, _, _, _ = tasks[t]
                gath[slot] = pltpu.async_copy(
                    tab.at[idxs[t]], bufs[slot], sems_g[slot])
            if t >= lookahead:
                u = t - lookahead
                slot = u % nbuf
                gath[slot].wait()
                _, _, out, off = tasks[u]
                writes[slot] = pltpu.async_copy(
                    bufs[slot], out.at[pl.ds(base + off, ch)], sems_w[slot])
        for w in writes:
            if w is not None:
                w.wait()

    return gather_kernel(t0, t1, t2, i0, i1, i2)


_TCG = 2


def _tc_body(e0_r, e1_r, e2_r, w0_r, w1_r, w2_r, b_r, out_r):
    acc = jnp.dot(e0_r[...], w0_r[...], preferred_element_type=jnp.float32)
    acc += jnp.dot(e1_r[...], w1_r[...], preferred_element_type=jnp.float32)
    acc += jnp.dot(e2_r[...], w2_r[...], preferred_element_type=jnp.float32)
    out_r[...] = acc + b_r[...]


def _tc_body_alias(e0_r, e1_r, e2_r, w0_r, w1_r, w2_r, b_r, acc_r, out_r):
    del acc_r
    _tc_body(e0_r, e1_r, e2_r, w0_r, w1_r, w2_r, b_r, out_r)


def _tc_matmul_chunk(e0, e1, e2, w0, w1, w2, bsum, chunk, acc):
    bs = _CB // _TCG
    eb = pl.BlockSpec((bs, D), lambda i: (i, 0))
    wb = pl.BlockSpec((D, D), lambda i: (0, 0))
    bb = pl.BlockSpec((1, D), lambda i: (0, 0))
    out_spec = pl.BlockSpec((bs, D), lambda i: (chunk * _TCG + i, 0))
    params = pltpu.CompilerParams(dimension_semantics=("arbitrary",))
    if acc is None:
        return pl.pallas_call(
            _tc_body,
            grid=(_TCG,),
            in_specs=[eb, eb, eb, wb, wb, wb, bb],
            out_specs=out_spec,
            out_shape=jax.ShapeDtypeStruct((B, D), jnp.float32),
            compiler_params=params,
        )(e0, e1, e2, w0, w1, w2, bsum)
    return pl.pallas_call(
        _tc_body_alias,
        grid=(_TCG,),
        in_specs=[eb, eb, eb, wb, wb, wb, bb,
                  pl.BlockSpec(memory_space=pl.ANY)],
        out_specs=out_spec,
        out_shape=jax.ShapeDtypeStruct((B, D), jnp.float32),
        input_output_aliases={7: 0},
        compiler_params=params,
    )(e0, e1, e2, w0, w1, w2, bsum, acc)


def kernel(Target, Type, Location, T_Target, T_Type, T_Location,
           W0, b0, W1, b1, W2, b2):
    i0 = Target.astype(jnp.int32)
    i1 = Type.astype(jnp.int32)
    i2 = Location.astype(jnp.int32)
    bsum = (b0 + b1 + b2).reshape(1, D)
    chunks = []
    for c in range(_NCHUNKS):
        chunks.append(_sc_gather3(T_Target, T_Type, T_Location,
                                  i0, i1, i2, _CB, c * _CB))
    acc = None
    for c, (e0, e1, e2) in enumerate(chunks):
        acc = _tc_matmul_chunk(e0, e1, e2, W0, W1, W2, bsum, c, acc)
    return acc
